# baseline (speedup 1.0000x reference)
import functools

import jax
import jax.numpy as jnp
from jax import lax
from jax.experimental import pallas as pl
from jax.experimental.pallas import tpu as pltpu, tpu_sc as plsc

N = 50000
E = 800000
G = 64
NODE_IN = 128
EDGE_IN = 16
H = 64
NC_OUT = 5
NL = 3
BN_EPS = 1e-5

NP = 50176
EP = 802816
NBLK = 1024
EBLK = 2048
N_GRID = NP // NBLK
E_GRID = EP // EBLK
PBLK = 7168
P_GRID = NP // PBLK
HH = H // 2

NUM_TILES = 16
CHUNK = 128
CHUNKS_PER_TILE = EP // (NUM_TILES * CHUNK)
STEPS = CHUNKS_PER_TILE // 4
ROWS_PER_TILE = NP // NUM_TILES
EC = CHUNK * HH // 128


def _pack(y):
    ng = y.shape[0] // 1024
    return jnp.concatenate([
        jnp.concatenate([y[t * 1024 + q * 256:t * 1024 + (q + 1) * 256, :]
                         for q in range(4)], axis=1)
        for t in range(ng)], axis=0)


def _unpack(b):
    ng = b.shape[0] // 256
    return jnp.concatenate([
        b[t * 256:(t + 1) * 256, q * 32:(q + 1) * 32]
        for t in range(ng) for q in range(4)], axis=0)


def _proj_node_body(x_ref, w_ref, b_ref, h_ref):
    h = jnp.dot(x_ref[...], w_ref[...], preferred_element_type=jnp.float32)
    h = h + b_ref[...]
    h_ref[0] = _pack(h[:, :HH])
    h_ref[1] = _pack(h[:, HH:])


def _proj_node(x_p, node_w, node_b):
    return pl.pallas_call(
        _proj_node_body,
        grid=(N_GRID,),
        in_specs=[
            pl.BlockSpec((NBLK, NODE_IN), lambda i: (i, 0)),
            pl.BlockSpec((NODE_IN, H), lambda i: (0, 0)),
            pl.BlockSpec((1, H), lambda i: (0, 0)),
        ],
        out_specs=pl.BlockSpec((2, NBLK * HH // 128, 128), lambda i: (0, i, 0)),
        out_shape=jax.ShapeDtypeStruct((2, NP * HH // 128, 128), jnp.float32),
    )(x_p, node_w, node_b.reshape(1, H))


def _proj_edge_body(ea_ref, ew_ref, eb_ref, lw_ref, lb_ref,
                    e0_ref, e1_ref, e2_ref):
    wc = jnp.dot(ew_ref[...], lw_ref[...], preferred_element_type=jnp.float32)
    bc = jnp.dot(eb_ref[...], lw_ref[...],
                 preferred_element_type=jnp.float32) + lb_ref[...]
    x8 = ea_ref[...]
    ys = [jnp.dot(x8[:, 16 * j:16 * (j + 1)], wc,
                  preferred_element_type=jnp.float32) + bc for j in range(8)]
    for l, ref in enumerate((e0_ref, e1_ref, e2_ref)):
        for h in range(2):
            sl = slice(l * H + h * HH, l * H + (h + 1) * HH)
            ref[h] = jnp.concatenate(
                [jnp.concatenate([ys[4 * p + k][:, sl] for k in range(4)],
                                 axis=1) for p in range(2)], axis=0)


def _proj_edge(ea_lin, edge_w, edge_b, lw_all, lb_all):
    out_sds = jax.ShapeDtypeStruct((2, EP * HH // 128, 128), jnp.float32)
    return pl.pallas_call(
        _proj_edge_body,
        grid=(E_GRID,),
        in_specs=[
            pl.BlockSpec((EBLK * EDGE_IN // 128, 128), lambda i: (i, 0)),
            pl.BlockSpec((EDGE_IN, H), lambda i: (0, 0)),
            pl.BlockSpec((1, H), lambda i: (0, 0)),
            pl.BlockSpec((H, NL * H), lambda i: (0, 0)),
            pl.BlockSpec((1, NL * H), lambda i: (0, 0)),
        ],
        out_specs=[pl.BlockSpec((2, EBLK * HH // 128, 128),
                                lambda i: (0, i, 0))] * NL,
        out_shape=[out_sds] * NL,
    )(ea_lin, edge_w, edge_b.reshape(1, H), lw_all, lb_all)


def _sc_msg_body(h_hbm, ee_hbm, src_hbm, dst_hbm, aggr_hbm,
                 accum, sidx, didx, rows, eebuf, isem, gsem):
    c = lax.axis_index("c")
    s = lax.axis_index("s")
    tile_base = s * ROWS_PER_TILE
    src_off = c * NP

    def zero_row(i, _):
        z = jnp.zeros((16,), jnp.float32)
        rows[0][i, pl.ds(0, 16)] = z
        rows[0][i, pl.ds(16, 16)] = z
        return 0
    lax.fori_loop(0, CHUNK, zero_row, 0)
    zc = 112
    for k in range(ROWS_PER_TILE // zc):
        pltpu.sync_copy(rows[0].at[pl.ds(0, zc)],
                        accum.at[pl.ds(tile_base + k * zc, zc)])
    plsc.subcore_barrier()

    chunk0 = s * CHUNKS_PER_TILE

    def issue_idx(slot, t):
        g = jnp.minimum(chunk0 + t, CHUNKS_PER_TILE * NUM_TILES - 1)
        pltpu.async_copy(src_hbm.at[pl.ds(g, 1)], sidx[slot], isem[slot])
        pltpu.async_copy(dst_hbm.at[pl.ds(g, 1)], didx[slot], isem[slot])

    def wait_idx(slot):
        pltpu.make_async_copy(src_hbm.at[pl.ds(0, 1)], sidx[slot],
                              isem[slot]).wait()
        pltpu.make_async_copy(dst_hbm.at[pl.ds(0, 1)], didx[slot],
                              isem[slot]).wait()

    def issue_gather(slot, islot, t):
        for k in range(8):
            sl = pl.ds(k * 16, 16)
            sidx[islot][0, sl] = sidx[islot][0, sl] + src_off
        pltpu.async_copy(h_hbm.at[sidx[islot].at[0]], rows[slot], gsem[slot])
        te = jnp.minimum(chunk0 + t, CHUNKS_PER_TILE * NUM_TILES - 1)
        pltpu.async_copy(ee_hbm.at[c, pl.ds(te * EC, EC)],
                         eebuf[slot], gsem[slot])

    def wait_gather(slot):
        pltpu.make_async_copy(h_hbm.at[pl.ds(0, CHUNK)], rows[slot],
                              gsem[slot]).wait()
        pltpu.make_async_copy(ee_hbm.at[0, pl.ds(0, EC)], eebuf[slot],
                              gsem[slot]).wait()

    def compute_scatter(slot, islot):
        @plsc.parallel_loop(0, EC, unroll=4)
        def msg_row(r):
            for j in range(8):
                rsl = pl.ds((j % 2) * 16, 16)
                ri = r * 4 + j // 2
                v = eebuf[slot][r, pl.ds(j * 16, 16)] + rows[slot][ri, rsl]
                rows[slot][ri, rsl] = jnp.maximum(v, 0.0)
        pltpu.sync_copy(rows[slot], accum.at[didx[islot].at[0]], add=True)

    NT = CHUNKS_PER_TILE
    for t in range(4):
        issue_idx(t, t)
    wait_idx(0)
    issue_gather(0, 0, 0)

    def step(i, _):
        t0 = 4 * i

        def stage(data_cur, data_nxt, islot_cur, islot_nxt, islot_refill, dt):
            wait_idx(islot_nxt)
            issue_gather(data_nxt, islot_nxt, t0 + dt + 1)
            wait_gather(data_cur)
            compute_scatter(data_cur, islot_cur)
            issue_idx(islot_refill, t0 + dt + 4)

        stage(0, 1, 0, 1, 0, 0)
        stage(1, 0, 1, 2, 1, 1)
        stage(0, 1, 2, 3, 2, 2)
        stage(1, 0, 3, 0, 3, 3)
        return 0

    lax.fori_loop(0, STEPS - 1, step, 0)

    tL = 4 * (STEPS - 1)
    wait_idx(1)
    issue_gather(1, 1, tL + 1)
    wait_gather(0)
    compute_scatter(0, 0)
    wait_idx(2)
    issue_gather(0, 2, tL + 2)
    wait_gather(1)
    compute_scatter(1, 1)
    wait_idx(3)
    issue_gather(1, 3, tL + 3)
    wait_gather(0)
    compute_scatter(0, 2)
    wait_gather(1)
    compute_scatter(1, 3)

    plsc.subcore_barrier()
    pltpu.sync_copy(accum.at[pl.ds(tile_base, ROWS_PER_TILE)],
                    aggr_hbm.at[c, pl.ds(tile_base, ROWS_PER_TILE)])


@jax.jit
def _sc_msg(h_flat, ee, src2d, dst2d):
    mesh = plsc.VectorSubcoreMesh(core_axis_name="c", subcore_axis_name="s",
                                  num_cores=2, num_subcores=NUM_TILES)
    f = functools.partial(
        pl.kernel,
        out_type=jax.ShapeDtypeStruct((2, NP, HH), jnp.float32),
        mesh=mesh,
        scratch_types=[
            pltpu.VMEM_SHARED((NP, HH), jnp.float32),
            [pltpu.VMEM((1, CHUNK), jnp.int32) for _ in range(4)],
            [pltpu.VMEM((1, CHUNK), jnp.int32) for _ in range(4)],
            [pltpu.VMEM((CHUNK, HH), jnp.float32) for _ in range(2)],
            [pltpu.VMEM((EC, 128), jnp.float32) for _ in range(2)],
            [pltpu.SemaphoreType.DMA for _ in range(4)],
            [pltpu.SemaphoreType.DMA for _ in range(2)],
        ],
        compiler_params=pltpu.CompilerParams(use_tc_tiling_on_sc=False),
    )(_sc_msg_body)
    return f(h_flat, ee, src2d, dst2d)


def _mlp_body(h_ref, a_ref, w1_ref, b1_ref, w2_ref, b2_ref, sc_ref, sb_ref,
              out_ref):
    z = jnp.concatenate(
        [_unpack(h_ref[0] + a_ref[0]),
         _unpack(h_ref[1] + a_ref[1])], axis=1)
    t = jnp.maximum(
        jnp.dot(z, w1_ref[...], preferred_element_type=jnp.float32)
        + b1_ref[...], 0.0)
    t = jnp.dot(t, w2_ref[...], preferred_element_type=jnp.float32) + b2_ref[...]
    t = t * sc_ref[...] + sb_ref[...]
    t = jnp.maximum(t, 0.0)
    out_ref[0] = _pack(t[:, :HH])
    out_ref[1] = _pack(t[:, HH:])


def _node_mlp(h, aggr, w1, b1, w2, b2, scale, bias):
    wspec = pl.BlockSpec((H, H), lambda i: (0, 0))
    vspec = pl.BlockSpec((1, H), lambda i: (0, 0))
    lin_spec = pl.BlockSpec((2, NBLK * HH // 128, 128), lambda i: (0, i, 0))
    return pl.pallas_call(
        _mlp_body,
        grid=(N_GRID,),
        in_specs=[lin_spec, lin_spec, wspec, vspec, wspec, vspec, vspec, vspec],
        out_specs=lin_spec,
        out_shape=jax.ShapeDtypeStruct((2, NP * HH // 128, 128), jnp.float32),
    )(h, aggr, w1, b1.reshape(1, H), w2, b2.reshape(1, H),
      scale.reshape(1, H), bias.reshape(1, H))


def _pool_body(h_ref, bb_ref, w1_ref, b1_ref, w2_ref, b2_ref, out_ref,
               sums_ref, maxes_ref, counts_ref):
    i = pl.program_id(0)

    @pl.when(i == 0)
    def _init():
        sums_ref[...] = jnp.zeros_like(sums_ref)
        counts_ref[...] = jnp.zeros_like(counts_ref)
        maxes_ref[...] = jnp.full_like(maxes_ref, -1e30)

    hcat = jnp.concatenate([_unpack(h_ref[0]),
                            _unpack(h_ref[1])], axis=1)
    bb = bb_ref[...]
    oh = (bb == lax.broadcasted_iota(jnp.int32, bb.shape, 1)
          .astype(jnp.float32))
    oh = oh.astype(jnp.float32)
    sums_ref[...] += lax.dot_general(
        oh, hcat, (((0,), (0,)), ((), ())),
        preferred_element_type=jnp.float32)
    counts_ref[...] += jnp.sum(oh, axis=0, keepdims=True)
    bmin = jnp.min(bb).astype(jnp.int32)
    bmax = jnp.minimum(jnp.max(bb).astype(jnp.int32), G - 1)

    def seg_max(g, _):
        m = jnp.where(bb[:, :1] == g.astype(jnp.float32), hcat, -1e30)
        mg = jnp.max(m, axis=0, keepdims=True)
        maxes_ref[pl.ds(g, 1), :] = jnp.maximum(maxes_ref[pl.ds(g, 1), :], mg)
        return 0
    lax.fori_loop(bmin, bmax + 1, seg_max, 0)

    @pl.when(i == pl.num_programs(0) - 1)
    def _final():
        counts = counts_ref[...].reshape(G, 1)
        sums = sums_ref[...]
        mean = sums / jnp.maximum(counts, 1.0)
        hmax = jnp.where(counts > 0.0, maxes_ref[...], 0.0)
        feat = jnp.concatenate([mean, hmax, sums], axis=1)
        r = jnp.maximum(
            jnp.dot(feat, w1_ref[...], preferred_element_type=jnp.float32)
            + b1_ref[...], 0.0)
        out_ref[...] = (jnp.dot(r, w2_ref[...],
                                preferred_element_type=jnp.float32)
                        + b2_ref[...])


def _pool_head(h, bb, w1, b1, w2, b2):
    return pl.pallas_call(
        _pool_body,
        grid=(P_GRID,),
        in_specs=[
            pl.BlockSpec((2, PBLK * HH // 128, 128), lambda i: (0, i, 0)),
            pl.BlockSpec((PBLK, G), lambda i: (i, 0)),
            pl.BlockSpec((3 * H, H), lambda i: (0, 0)),
            pl.BlockSpec((1, H), lambda i: (0, 0)),
            pl.BlockSpec((H, NC_OUT), lambda i: (0, 0)),
            pl.BlockSpec((1, NC_OUT), lambda i: (0, 0)),
        ],
        out_specs=pl.BlockSpec((G, NC_OUT), lambda i: (0, 0)),
        out_shape=jax.ShapeDtypeStruct((G, NC_OUT), jnp.float32),
        scratch_shapes=[
            pltpu.VMEM((G, H), jnp.float32),
            pltpu.VMEM((G, H), jnp.float32),
            pltpu.VMEM((1, G), jnp.float32),
        ],
    )(h, bb, w1, b1.reshape(1, H), w2, b2.reshape(1, NC_OUT))


def _id2flat_vals(v):
    return (v & ~1023) | ((v & 255) << 2) | ((v >> 8) & 3)


def _to_flat_order(a):
    return a.reshape(-1, 256, 2, 4).transpose(0, 2, 1, 3).reshape(a.shape)


def kernel(x, edge_attr, edge_index, batch, params):
    x_p = jnp.pad(x, ((0, NP - N), (0, 0)))
    ea_lin = jnp.pad(edge_attr.reshape(E * EDGE_IN // 128, 128),
                     ((0, (EP - E) * EDGE_IN // 128), (0, 0)))
    src_f = _to_flat_order(_id2flat_vals(jnp.pad(edge_index[0], (0, EP - E))))
    dst_f = _to_flat_order(_id2flat_vals(
        jnp.pad(edge_index[1], (0, EP - E), constant_values=N)))
    src2d = src_f.reshape(EP // CHUNK, CHUNK)
    dst2d = dst_f.reshape(EP // CHUNK, CHUNK)
    bb = jnp.broadcast_to(
        jnp.pad(batch, (0, NP - N), constant_values=G)[:, None].astype(
            jnp.float32), (NP, G))

    lw_all = jnp.concatenate([lp['lin_e_w'] for lp in params['layers']], axis=1)
    lb_all = jnp.concatenate([lp['lin_e_b'] for lp in params['layers']]
                             ).reshape(1, NL * H)

    h = _proj_node(x_p, params['node_w'], params['node_b'])
    ees = _proj_edge(ea_lin, params['edge_w'], params['edge_b'],
                     lw_all, lb_all)

    inv = 1.0 / jnp.sqrt(1.0 + BN_EPS)
    for l, lp in enumerate(params['layers']):
        aggr = _sc_msg(h.reshape(2 * NP, HH), ees[l], src2d, dst2d)
        h = _node_mlp(h, aggr.reshape(2, NP * HH // 128, 128),
                      lp['mlp_w1'], lp['mlp_b1'],
                      lp['mlp_w2'], lp['mlp_b2'],
                      lp['bn_g'] * inv, lp['bn_b'])

    return _pool_head(h, bb, params['head_w1'], params['head_b1'],
                      params['head_w2'], params['head_b2'])


# speedup vs baseline: 1.1814x; 1.1814x over previous
import functools

import jax
import jax.numpy as jnp
from jax import lax
from jax.experimental import pallas as pl
from jax.experimental.pallas import tpu as pltpu, tpu_sc as plsc

N = 50000
E = 800000
G = 64
NODE_IN = 128
EDGE_IN = 16
H = 64
NC_OUT = 5
NL = 3
BN_EPS = 1e-5

NP = 50176
EP = 802816
NBLK = 1024
EBLK = 2048
N_GRID = NP // NBLK
E_GRID = EP // EBLK
PBLK = 7168
P_GRID = NP // PBLK
HH = H // 2

NUM_TILES = 16
CHUNK = 128
CHUNKS_PER_TILE = EP // (NUM_TILES * CHUNK)
STEPS = CHUNKS_PER_TILE // 4
ROWS_PER_TILE = NP // NUM_TILES
EC = CHUNK * HH // 128


def _pack(y):
    ng = y.shape[0] // 1024
    return jnp.concatenate([
        jnp.concatenate([y[t * 1024 + q * 256:t * 1024 + (q + 1) * 256, :]
                         for q in range(4)], axis=1)
        for t in range(ng)], axis=0)


def _unpack(b):
    ng = b.shape[0] // 256
    return jnp.concatenate([
        b[t * 256:(t + 1) * 256, q * 32:(q + 1) * 32]
        for t in range(ng) for q in range(4)], axis=0)


def _proj_node_body(x_ref, w_ref, b_ref, h_ref):
    h = jnp.dot(x_ref[...], w_ref[...], preferred_element_type=jnp.float32)
    h = h + b_ref[...]
    h_ref[0] = _pack(h[:, :HH])
    h_ref[1] = _pack(h[:, HH:])


def _proj_node(x_p, node_w, node_b):
    return pl.pallas_call(
        _proj_node_body,
        grid=(N_GRID,),
        in_specs=[
            pl.BlockSpec((NBLK, NODE_IN), lambda i: (i, 0)),
            pl.BlockSpec((NODE_IN, H), lambda i: (0, 0)),
            pl.BlockSpec((1, H), lambda i: (0, 0)),
        ],
        out_specs=pl.BlockSpec((2, NBLK * HH // 128, 128), lambda i: (0, i, 0)),
        out_shape=jax.ShapeDtypeStruct((2, NP * HH // 128, 128), jnp.float32),
    )(x_p, node_w, node_b.reshape(1, H))


def _proj_edge_body(ea_ref, ew_ref, eb_ref, lw_ref, lb_ref,
                    e0_ref, e1_ref, e2_ref):
    wc = jnp.dot(ew_ref[...], lw_ref[...], preferred_element_type=jnp.float32)
    bc = jnp.dot(eb_ref[...], lw_ref[...],
                 preferred_element_type=jnp.float32) + lb_ref[...]
    ee = jnp.dot(ea_ref[...], wc, preferred_element_type=jnp.float32) + bc
    for l, ref in enumerate((e0_ref, e1_ref, e2_ref)):
        ref[0] = _pack(ee[:, l * H:l * H + HH])
        ref[1] = _pack(ee[:, l * H + HH:(l + 1) * H])


def _proj_edge(ea_lin, edge_w, edge_b, lw_all, lb_all):
    out_sds = jax.ShapeDtypeStruct((2, EP * HH // 128, 128), jnp.float32)
    return pl.pallas_call(
        _proj_edge_body,
        grid=(E_GRID,),
        in_specs=[
            pl.BlockSpec((EBLK, EDGE_IN), lambda i: (i, 0)),
            pl.BlockSpec((EDGE_IN, H), lambda i: (0, 0)),
            pl.BlockSpec((1, H), lambda i: (0, 0)),
            pl.BlockSpec((H, NL * H), lambda i: (0, 0)),
            pl.BlockSpec((1, NL * H), lambda i: (0, 0)),
        ],
        out_specs=[pl.BlockSpec((2, EBLK * HH // 128, 128),
                                lambda i: (0, i, 0))] * NL,
        out_shape=[out_sds] * NL,
    )(ea_lin, edge_w, edge_b.reshape(1, H), lw_all, lb_all)


def _sc_msg_body(h_hbm, ee_hbm, src_hbm, dst_hbm, aggr_hbm,
                 accum, sidx, didx, rows, eebuf, isem, gsem):
    c = lax.axis_index("c")
    s = lax.axis_index("s")
    tile_base = s * ROWS_PER_TILE
    src_off = c * NP

    def zero_row(i, _):
        z = jnp.zeros((16,), jnp.float32)
        rows[0][i, pl.ds(0, 16)] = z
        rows[0][i, pl.ds(16, 16)] = z
        return 0
    lax.fori_loop(0, CHUNK, zero_row, 0)
    zc = 112
    for k in range(ROWS_PER_TILE // zc):
        pltpu.sync_copy(rows[0].at[pl.ds(0, zc)],
                        accum.at[pl.ds(tile_base + k * zc, zc)])
    plsc.subcore_barrier()

    chunk0 = s * CHUNKS_PER_TILE

    def issue_idx(slot, t):
        g = jnp.minimum(chunk0 + t, CHUNKS_PER_TILE * NUM_TILES - 1)
        pltpu.async_copy(src_hbm.at[pl.ds(g, 1)], sidx[slot], isem[slot])
        pltpu.async_copy(dst_hbm.at[pl.ds(g, 1)], didx[slot], isem[slot])

    def wait_idx(slot):
        pltpu.make_async_copy(src_hbm.at[pl.ds(0, 1)], sidx[slot],
                              isem[slot]).wait()
        pltpu.make_async_copy(dst_hbm.at[pl.ds(0, 1)], didx[slot],
                              isem[slot]).wait()

    def issue_gather(slot, islot, t):
        for k in range(8):
            sl = pl.ds(k * 16, 16)
            sidx[islot][0, sl] = sidx[islot][0, sl] + src_off
        pltpu.async_copy(h_hbm.at[sidx[islot].at[0]], rows[slot], gsem[slot])
        te = jnp.minimum(chunk0 + t, CHUNKS_PER_TILE * NUM_TILES - 1)
        pltpu.async_copy(ee_hbm.at[c, pl.ds(te * EC, EC)],
                         eebuf[slot], gsem[slot])

    def wait_gather(slot):
        pltpu.make_async_copy(h_hbm.at[pl.ds(0, CHUNK)], rows[slot],
                              gsem[slot]).wait()
        pltpu.make_async_copy(ee_hbm.at[0, pl.ds(0, EC)], eebuf[slot],
                              gsem[slot]).wait()

    def compute_scatter(slot, islot):
        @plsc.parallel_loop(0, EC, unroll=4)
        def msg_row(r):
            for j in range(8):
                rsl = pl.ds((j % 2) * 16, 16)
                ri = r * 4 + j // 2
                v = eebuf[slot][r, pl.ds(j * 16, 16)] + rows[slot][ri, rsl]
                rows[slot][ri, rsl] = jnp.maximum(v, 0.0)
        pltpu.sync_copy(rows[slot], accum.at[didx[islot].at[0]], add=True)

    NT = CHUNKS_PER_TILE
    for t in range(4):
        issue_idx(t, t)
    wait_idx(0)
    issue_gather(0, 0, 0)

    def step(i, _):
        t0 = 4 * i

        def stage(data_cur, data_nxt, islot_cur, islot_nxt, islot_refill, dt):
            wait_idx(islot_nxt)
            issue_gather(data_nxt, islot_nxt, t0 + dt + 1)
            wait_gather(data_cur)
            compute_scatter(data_cur, islot_cur)
            issue_idx(islot_refill, t0 + dt + 4)

        stage(0, 1, 0, 1, 0, 0)
        stage(1, 0, 1, 2, 1, 1)
        stage(0, 1, 2, 3, 2, 2)
        stage(1, 0, 3, 0, 3, 3)
        return 0

    lax.fori_loop(0, STEPS - 1, step, 0)

    tL = 4 * (STEPS - 1)
    wait_idx(1)
    issue_gather(1, 1, tL + 1)
    wait_gather(0)
    compute_scatter(0, 0)
    wait_idx(2)
    issue_gather(0, 2, tL + 2)
    wait_gather(1)
    compute_scatter(1, 1)
    wait_idx(3)
    issue_gather(1, 3, tL + 3)
    wait_gather(0)
    compute_scatter(0, 2)
    wait_gather(1)
    compute_scatter(1, 3)

    plsc.subcore_barrier()
    pltpu.sync_copy(accum.at[pl.ds(tile_base, ROWS_PER_TILE)],
                    aggr_hbm.at[c, pl.ds(tile_base, ROWS_PER_TILE)])


@jax.jit
def _sc_msg(h_flat, ee, src2d, dst2d):
    mesh = plsc.VectorSubcoreMesh(core_axis_name="c", subcore_axis_name="s",
                                  num_cores=2, num_subcores=NUM_TILES)
    f = functools.partial(
        pl.kernel,
        out_type=jax.ShapeDtypeStruct((2, NP, HH), jnp.float32),
        mesh=mesh,
        scratch_types=[
            pltpu.VMEM_SHARED((NP, HH), jnp.float32),
            [pltpu.VMEM((1, CHUNK), jnp.int32) for _ in range(4)],
            [pltpu.VMEM((1, CHUNK), jnp.int32) for _ in range(4)],
            [pltpu.VMEM((CHUNK, HH), jnp.float32) for _ in range(2)],
            [pltpu.VMEM((EC, 128), jnp.float32) for _ in range(2)],
            [pltpu.SemaphoreType.DMA for _ in range(4)],
            [pltpu.SemaphoreType.DMA for _ in range(2)],
        ],
        compiler_params=pltpu.CompilerParams(use_tc_tiling_on_sc=False),
    )(_sc_msg_body)
    return f(h_flat, ee, src2d, dst2d)


def _mlp_body(h_ref, a_ref, w1_ref, b1_ref, w2_ref, b2_ref, sc_ref, sb_ref,
              out_ref):
    z = jnp.concatenate(
        [_unpack(h_ref[0] + a_ref[0]),
         _unpack(h_ref[1] + a_ref[1])], axis=1)
    t = jnp.maximum(
        jnp.dot(z, w1_ref[...], preferred_element_type=jnp.float32)
        + b1_ref[...], 0.0)
    t = jnp.dot(t, w2_ref[...], preferred_element_type=jnp.float32) + b2_ref[...]
    t = t * sc_ref[...] + sb_ref[...]
    t = jnp.maximum(t, 0.0)
    out_ref[0] = _pack(t[:, :HH])
    out_ref[1] = _pack(t[:, HH:])


def _node_mlp(h, aggr, w1, b1, w2, b2, scale, bias):
    wspec = pl.BlockSpec((H, H), lambda i: (0, 0))
    vspec = pl.BlockSpec((1, H), lambda i: (0, 0))
    lin_spec = pl.BlockSpec((2, NBLK * HH // 128, 128), lambda i: (0, i, 0))
    return pl.pallas_call(
        _mlp_body,
        grid=(N_GRID,),
        in_specs=[lin_spec, lin_spec, wspec, vspec, wspec, vspec, vspec, vspec],
        out_specs=lin_spec,
        out_shape=jax.ShapeDtypeStruct((2, NP * HH // 128, 128), jnp.float32),
    )(h, aggr, w1, b1.reshape(1, H), w2, b2.reshape(1, H),
      scale.reshape(1, H), bias.reshape(1, H))


def _pool_body(h_ref, bb_ref, w1_ref, b1_ref, w2_ref, b2_ref, out_ref,
               sums_ref, maxes_ref, counts_ref):
    i = pl.program_id(0)

    @pl.when(i == 0)
    def _init():
        sums_ref[...] = jnp.zeros_like(sums_ref)
        counts_ref[...] = jnp.zeros_like(counts_ref)
        maxes_ref[...] = jnp.full_like(maxes_ref, -1e30)

    hcat = jnp.concatenate([_unpack(h_ref[0]),
                            _unpack(h_ref[1])], axis=1)
    bb = bb_ref[...]
    oh = (bb == lax.broadcasted_iota(jnp.int32, bb.shape, 1)
          .astype(jnp.float32))
    oh = oh.astype(jnp.float32)
    sums_ref[...] += lax.dot_general(
        oh, hcat, (((0,), (0,)), ((), ())),
        preferred_element_type=jnp.float32)
    counts_ref[...] += jnp.sum(oh, axis=0, keepdims=True)
    bmin = jnp.min(bb).astype(jnp.int32)
    bmax = jnp.minimum(jnp.max(bb).astype(jnp.int32), G - 1)

    def seg_max(g, _):
        m = jnp.where(bb[:, :1] == g.astype(jnp.float32), hcat, -1e30)
        mg = jnp.max(m, axis=0, keepdims=True)
        maxes_ref[pl.ds(g, 1), :] = jnp.maximum(maxes_ref[pl.ds(g, 1), :], mg)
        return 0
    lax.fori_loop(bmin, bmax + 1, seg_max, 0)

    @pl.when(i == pl.num_programs(0) - 1)
    def _final():
        counts = counts_ref[...].reshape(G, 1)
        sums = sums_ref[...]
        mean = sums / jnp.maximum(counts, 1.0)
        hmax = jnp.where(counts > 0.0, maxes_ref[...], 0.0)
        feat = jnp.concatenate([mean, hmax, sums], axis=1)
        r = jnp.maximum(
            jnp.dot(feat, w1_ref[...], preferred_element_type=jnp.float32)
            + b1_ref[...], 0.0)
        out_ref[...] = (jnp.dot(r, w2_ref[...],
                                preferred_element_type=jnp.float32)
                        + b2_ref[...])


def _pool_head(h, bb, w1, b1, w2, b2):
    return pl.pallas_call(
        _pool_body,
        grid=(P_GRID,),
        in_specs=[
            pl.BlockSpec((2, PBLK * HH // 128, 128), lambda i: (0, i, 0)),
            pl.BlockSpec((PBLK, G), lambda i: (i, 0)),
            pl.BlockSpec((3 * H, H), lambda i: (0, 0)),
            pl.BlockSpec((1, H), lambda i: (0, 0)),
            pl.BlockSpec((H, NC_OUT), lambda i: (0, 0)),
            pl.BlockSpec((1, NC_OUT), lambda i: (0, 0)),
        ],
        out_specs=pl.BlockSpec((G, NC_OUT), lambda i: (0, 0)),
        out_shape=jax.ShapeDtypeStruct((G, NC_OUT), jnp.float32),
        scratch_shapes=[
            pltpu.VMEM((G, H), jnp.float32),
            pltpu.VMEM((G, H), jnp.float32),
            pltpu.VMEM((1, G), jnp.float32),
        ],
    )(h, bb, w1, b1.reshape(1, H), w2, b2.reshape(1, NC_OUT))


def _id2flat_vals(v):
    return (v & ~1023) | ((v & 255) << 2) | ((v >> 8) & 3)


def _to_flat_order(a):
    return a.reshape(-1, 4, 256).transpose(0, 2, 1).reshape(a.shape)


def kernel(x, edge_attr, edge_index, batch, params):
    x_p = jnp.pad(x, ((0, NP - N), (0, 0)))
    ea_lin = jnp.pad(edge_attr, ((0, EP - E), (0, 0)))
    src_f = _to_flat_order(_id2flat_vals(jnp.pad(edge_index[0], (0, EP - E))))
    dst_f = _to_flat_order(_id2flat_vals(
        jnp.pad(edge_index[1], (0, EP - E), constant_values=N)))
    src2d = src_f.reshape(EP // CHUNK, CHUNK)
    dst2d = dst_f.reshape(EP // CHUNK, CHUNK)
    bb = jnp.broadcast_to(
        jnp.pad(batch, (0, NP - N), constant_values=G)[:, None].astype(
            jnp.float32), (NP, G))

    lw_all = jnp.concatenate([lp['lin_e_w'] for lp in params['layers']], axis=1)
    lb_all = jnp.concatenate([lp['lin_e_b'] for lp in params['layers']]
                             ).reshape(1, NL * H)

    h = _proj_node(x_p, params['node_w'], params['node_b'])
    ees = _proj_edge(ea_lin, params['edge_w'], params['edge_b'],
                     lw_all, lb_all)

    inv = 1.0 / jnp.sqrt(1.0 + BN_EPS)
    for l, lp in enumerate(params['layers']):
        aggr = _sc_msg(h.reshape(2 * NP, HH), ees[l], src2d, dst2d)
        h = _node_mlp(h, aggr.reshape(2, NP * HH // 128, 128),
                      lp['mlp_w1'], lp['mlp_b1'],
                      lp['mlp_w2'], lp['mlp_b2'],
                      lp['bn_g'] * inv, lp['bn_b'])

    return _pool_head(h, bb, params['head_w1'], params['head_b1'],
                      params['head_w2'], params['head_b2'])


# speedup vs baseline: 1.2424x; 1.0517x over previous
import functools

import jax
import jax.numpy as jnp
from jax import lax
from jax.experimental import pallas as pl
from jax.experimental.pallas import tpu as pltpu, tpu_sc as plsc

N = 50000
E = 800000
G = 64
NODE_IN = 128
EDGE_IN = 16
H = 64
NC_OUT = 5
NL = 3
BN_EPS = 1e-5

NP = 50176
EP = 802816
NBLK = 1024
EBLK = 4096
N_GRID = NP // NBLK
E_GRID = EP // EBLK
PBLK = 7168
P_GRID = NP // PBLK
HH = H // 2

NUM_TILES = 16
CHUNK = 128
CHUNKS_PER_TILE = EP // (NUM_TILES * CHUNK)
STEPS = CHUNKS_PER_TILE // 4
ROWS_PER_TILE = NP // NUM_TILES
EC = CHUNK * HH // 128


def _pack(y):
    ng = y.shape[0] // 1024
    return jnp.concatenate([
        jnp.concatenate([y[t * 1024 + q * 256:t * 1024 + (q + 1) * 256, :]
                         for q in range(4)], axis=1)
        for t in range(ng)], axis=0)


def _unpack(b):
    ng = b.shape[0] // 256
    return jnp.concatenate([
        b[t * 256:(t + 1) * 256, q * 32:(q + 1) * 32]
        for t in range(ng) for q in range(4)], axis=0)


def _proj_node_body(x_ref, w_ref, b_ref, h_ref):
    h = jnp.dot(x_ref[...], w_ref[...], preferred_element_type=jnp.float32)
    h = h + b_ref[...]
    h_ref[0] = _pack(h[:, :HH])
    h_ref[1] = _pack(h[:, HH:])


def _proj_node(x_p, node_w, node_b):
    return pl.pallas_call(
        _proj_node_body,
        grid=(N_GRID,),
        in_specs=[
            pl.BlockSpec((NBLK, NODE_IN), lambda i: (i, 0)),
            pl.BlockSpec((NODE_IN, H), lambda i: (0, 0)),
            pl.BlockSpec((1, H), lambda i: (0, 0)),
        ],
        out_specs=pl.BlockSpec((2, NBLK * HH // 128, 128), lambda i: (0, i, 0)),
        out_shape=jax.ShapeDtypeStruct((2, NP * HH // 128, 128), jnp.float32),
    )(x_p, node_w, node_b.reshape(1, H))


def _proj_edge_body(ea_ref, ew_ref, eb_ref, lw_ref, lb_ref,
                    e0_ref, e1_ref, e2_ref):
    wc = jnp.dot(ew_ref[...], lw_ref[...], preferred_element_type=jnp.float32)
    bc = jnp.dot(eb_ref[...], lw_ref[...],
                 preferred_element_type=jnp.float32) + lb_ref[...]
    ee = jnp.dot(ea_ref[...], wc, preferred_element_type=jnp.float32) + bc
    for l, ref in enumerate((e0_ref, e1_ref, e2_ref)):
        ref[0] = _pack(ee[:, l * H:l * H + HH])
        ref[1] = _pack(ee[:, l * H + HH:(l + 1) * H])


def _proj_edge(ea_lin, edge_w, edge_b, lw_all, lb_all):
    out_sds = jax.ShapeDtypeStruct((2, EP * HH // 128, 128), jnp.float32)
    return pl.pallas_call(
        _proj_edge_body,
        grid=(E_GRID,),
        in_specs=[
            pl.BlockSpec((EBLK, EDGE_IN), lambda i: (i, 0)),
            pl.BlockSpec((EDGE_IN, H), lambda i: (0, 0)),
            pl.BlockSpec((1, H), lambda i: (0, 0)),
            pl.BlockSpec((H, NL * H), lambda i: (0, 0)),
            pl.BlockSpec((1, NL * H), lambda i: (0, 0)),
        ],
        out_specs=[pl.BlockSpec((2, EBLK * HH // 128, 128),
                                lambda i: (0, i, 0))] * NL,
        out_shape=[out_sds] * NL,
    )(ea_lin, edge_w, edge_b.reshape(1, H), lw_all, lb_all)


def _sc_msg_body(h_hbm, ee_hbm, src_hbm, dst_hbm, aggr_hbm,
                 accum, sidx, didx, rows, eebuf, isem, gsem):
    c = lax.axis_index("c")
    s = lax.axis_index("s")
    tile_base = s * ROWS_PER_TILE
    src_off = c * NP

    def zero_row(i, _):
        z = jnp.zeros((16,), jnp.float32)
        rows[0][i, pl.ds(0, 16)] = z
        rows[0][i, pl.ds(16, 16)] = z
        return 0
    lax.fori_loop(0, CHUNK, zero_row, 0)
    zc = 112
    for k in range(ROWS_PER_TILE // zc):
        pltpu.sync_copy(rows[0].at[pl.ds(0, zc)],
                        accum.at[pl.ds(tile_base + k * zc, zc)])
    plsc.subcore_barrier()

    chunk0 = s * CHUNKS_PER_TILE

    def issue_idx(slot, t):
        g = jnp.minimum(chunk0 + t, CHUNKS_PER_TILE * NUM_TILES - 1)
        pltpu.async_copy(src_hbm.at[pl.ds(g, 1)], sidx[slot], isem[slot])
        pltpu.async_copy(dst_hbm.at[pl.ds(g, 1)], didx[slot], isem[slot])

    def wait_idx(slot):
        pltpu.make_async_copy(src_hbm.at[pl.ds(0, 1)], sidx[slot],
                              isem[slot]).wait()
        pltpu.make_async_copy(dst_hbm.at[pl.ds(0, 1)], didx[slot],
                              isem[slot]).wait()

    def issue_gather(slot, islot, t):
        for k in range(8):
            sl = pl.ds(k * 16, 16)
            sidx[islot][0, sl] = sidx[islot][0, sl] + src_off
        pltpu.async_copy(h_hbm.at[sidx[islot].at[0]], rows[slot], gsem[slot])
        te = jnp.minimum(chunk0 + t, CHUNKS_PER_TILE * NUM_TILES - 1)
        pltpu.async_copy(ee_hbm.at[c, pl.ds(te * EC, EC)],
                         eebuf[slot], gsem[slot])

    def wait_gather(slot):
        pltpu.make_async_copy(h_hbm.at[pl.ds(0, CHUNK)], rows[slot],
                              gsem[slot]).wait()
        pltpu.make_async_copy(ee_hbm.at[0, pl.ds(0, EC)], eebuf[slot],
                              gsem[slot]).wait()

    def compute_scatter(slot, islot):
        @plsc.parallel_loop(0, EC, unroll=4)
        def msg_row(r):
            for j in range(8):
                rsl = pl.ds((j % 2) * 16, 16)
                ri = r * 4 + j // 2
                v = eebuf[slot][r, pl.ds(j * 16, 16)] + rows[slot][ri, rsl]
                rows[slot][ri, rsl] = jnp.maximum(v, 0.0)
        pltpu.sync_copy(rows[slot], accum.at[didx[islot].at[0]], add=True)

    NT = CHUNKS_PER_TILE
    for t in range(4):
        issue_idx(t, t)
    wait_idx(0)
    issue_gather(0, 0, 0)

    def step(i, _):
        t0 = 4 * i

        def stage(data_cur, data_nxt, islot_cur, islot_nxt, islot_refill, dt):
            wait_idx(islot_nxt)
            issue_gather(data_nxt, islot_nxt, t0 + dt + 1)
            wait_gather(data_cur)
            compute_scatter(data_cur, islot_cur)
            issue_idx(islot_refill, t0 + dt + 4)

        stage(0, 1, 0, 1, 0, 0)
        stage(1, 0, 1, 2, 1, 1)
        stage(0, 1, 2, 3, 2, 2)
        stage(1, 0, 3, 0, 3, 3)
        return 0

    lax.fori_loop(0, STEPS - 1, step, 0)

    tL = 4 * (STEPS - 1)
    wait_idx(1)
    issue_gather(1, 1, tL + 1)
    wait_gather(0)
    compute_scatter(0, 0)
    wait_idx(2)
    issue_gather(0, 2, tL + 2)
    wait_gather(1)
    compute_scatter(1, 1)
    wait_idx(3)
    issue_gather(1, 3, tL + 3)
    wait_gather(0)
    compute_scatter(0, 2)
    wait_gather(1)
    compute_scatter(1, 3)

    plsc.subcore_barrier()
    pltpu.sync_copy(accum.at[pl.ds(tile_base, ROWS_PER_TILE)],
                    aggr_hbm.at[c, pl.ds(tile_base, ROWS_PER_TILE)])


@jax.jit
def _sc_msg(h_flat, ee, src2d, dst2d):
    mesh = plsc.VectorSubcoreMesh(core_axis_name="c", subcore_axis_name="s",
                                  num_cores=2, num_subcores=NUM_TILES)
    f = functools.partial(
        pl.kernel,
        out_type=jax.ShapeDtypeStruct((2, NP, HH), jnp.float32),
        mesh=mesh,
        scratch_types=[
            pltpu.VMEM_SHARED((NP, HH), jnp.float32),
            [pltpu.VMEM((1, CHUNK), jnp.int32) for _ in range(4)],
            [pltpu.VMEM((1, CHUNK), jnp.int32) for _ in range(4)],
            [pltpu.VMEM((CHUNK, HH), jnp.float32) for _ in range(2)],
            [pltpu.VMEM((EC, 128), jnp.float32) for _ in range(2)],
            [pltpu.SemaphoreType.DMA for _ in range(4)],
            [pltpu.SemaphoreType.DMA for _ in range(2)],
        ],
        compiler_params=pltpu.CompilerParams(use_tc_tiling_on_sc=False),
    )(_sc_msg_body)
    return f(h_flat, ee, src2d, dst2d)


def _mlp_body(h_ref, a_ref, w1_ref, b1_ref, w2_ref, b2_ref, sc_ref, sb_ref,
              out_ref):
    z = jnp.concatenate(
        [_unpack(h_ref[0] + a_ref[0]),
         _unpack(h_ref[1] + a_ref[1])], axis=1)
    t = jnp.maximum(
        jnp.dot(z, w1_ref[...], preferred_element_type=jnp.float32)
        + b1_ref[...], 0.0)
    t = jnp.dot(t, w2_ref[...], preferred_element_type=jnp.float32) + b2_ref[...]
    t = t * sc_ref[...] + sb_ref[...]
    t = jnp.maximum(t, 0.0)
    out_ref[0] = _pack(t[:, :HH])
    out_ref[1] = _pack(t[:, HH:])


def _node_mlp(h, aggr, w1, b1, w2, b2, scale, bias):
    wspec = pl.BlockSpec((H, H), lambda i: (0, 0))
    vspec = pl.BlockSpec((1, H), lambda i: (0, 0))
    lin_spec = pl.BlockSpec((2, NBLK * HH // 128, 128), lambda i: (0, i, 0))
    return pl.pallas_call(
        _mlp_body,
        grid=(N_GRID,),
        in_specs=[lin_spec, lin_spec, wspec, vspec, wspec, vspec, vspec, vspec],
        out_specs=lin_spec,
        out_shape=jax.ShapeDtypeStruct((2, NP * HH // 128, 128), jnp.float32),
    )(h, aggr, w1, b1.reshape(1, H), w2, b2.reshape(1, H),
      scale.reshape(1, H), bias.reshape(1, H))


def _pool_body(h_ref, bb_ref, w1_ref, b1_ref, w2_ref, b2_ref, out_ref,
               sums_ref, maxes_ref, counts_ref):
    i = pl.program_id(0)

    @pl.when(i == 0)
    def _init():
        sums_ref[...] = jnp.zeros_like(sums_ref)
        counts_ref[...] = jnp.zeros_like(counts_ref)
        maxes_ref[...] = jnp.full_like(maxes_ref, -1e30)

    hcat = jnp.concatenate([_unpack(h_ref[0]),
                            _unpack(h_ref[1])], axis=1)
    bb = bb_ref[...][:, :G]
    oh = (bb == lax.broadcasted_iota(jnp.int32, bb.shape, 1)
          .astype(jnp.float32))
    oh = oh.astype(jnp.float32)
    sums_ref[...] += lax.dot_general(
        oh, hcat, (((0,), (0,)), ((), ())),
        preferred_element_type=jnp.float32)
    counts_ref[...] += jnp.sum(oh, axis=0, keepdims=True)
    bmin = jnp.min(bb).astype(jnp.int32)
    bmax = jnp.minimum(jnp.max(bb).astype(jnp.int32), G - 1)

    def seg_max(g, _):
        m = jnp.where(bb[:, :1] == g.astype(jnp.float32), hcat, -1e30)
        mg = jnp.max(m, axis=0, keepdims=True)
        maxes_ref[pl.ds(g, 1), :] = jnp.maximum(maxes_ref[pl.ds(g, 1), :], mg)
        return 0
    lax.fori_loop(bmin, bmax + 1, seg_max, 0)

    @pl.when(i == pl.num_programs(0) - 1)
    def _final():
        counts = counts_ref[...].reshape(G, 1)
        sums = sums_ref[...]
        mean = sums / jnp.maximum(counts, 1.0)
        hmax = jnp.where(counts > 0.0, maxes_ref[...], 0.0)
        feat = jnp.concatenate([mean, hmax, sums], axis=1)
        r = jnp.maximum(
            jnp.dot(feat, w1_ref[...], preferred_element_type=jnp.float32)
            + b1_ref[...], 0.0)
        out_ref[...] = (jnp.dot(r, w2_ref[...],
                                preferred_element_type=jnp.float32)
                        + b2_ref[...])


def _pool_head(h, bb, w1, b1, w2, b2):
    return pl.pallas_call(
        _pool_body,
        grid=(P_GRID,),
        in_specs=[
            pl.BlockSpec((2, PBLK * HH // 128, 128), lambda i: (0, i, 0)),
            pl.BlockSpec((PBLK, 128), lambda i: (i, 0)),
            pl.BlockSpec((3 * H, H), lambda i: (0, 0)),
            pl.BlockSpec((1, H), lambda i: (0, 0)),
            pl.BlockSpec((H, NC_OUT), lambda i: (0, 0)),
            pl.BlockSpec((1, NC_OUT), lambda i: (0, 0)),
        ],
        out_specs=pl.BlockSpec((G, NC_OUT), lambda i: (0, 0)),
        out_shape=jax.ShapeDtypeStruct((G, NC_OUT), jnp.float32),
        scratch_shapes=[
            pltpu.VMEM((G, H), jnp.float32),
            pltpu.VMEM((G, H), jnp.float32),
            pltpu.VMEM((1, G), jnp.float32),
        ],
    )(h, bb, w1, b1.reshape(1, H), w2, b2.reshape(1, NC_OUT))


def _id2flat_vals(v):
    return (v & ~1023) | ((v & 255) << 2) | ((v >> 8) & 3)


def _to_flat_order(a):
    return a.reshape(-1, 4, 256).transpose(0, 2, 1).reshape(a.shape)


def kernel(x, edge_attr, edge_index, batch, params):
    x_p = jnp.pad(x, ((0, NP - N), (0, 0)))
    ea_lin = jnp.pad(edge_attr, ((0, EP - E), (0, 0)))
    src_f = _to_flat_order(_id2flat_vals(jnp.pad(edge_index[0], (0, EP - E))))
    dst_f = _to_flat_order(_id2flat_vals(
        jnp.pad(edge_index[1], (0, EP - E), constant_values=N)))
    src2d = src_f.reshape(EP // CHUNK, CHUNK)
    dst2d = dst_f.reshape(EP // CHUNK, CHUNK)
    bb = jnp.broadcast_to(
        jnp.pad(batch, (0, NP - N), constant_values=G)[:, None].astype(
            jnp.float32), (NP, 128))

    lw_all = jnp.concatenate([lp['lin_e_w'] for lp in params['layers']], axis=1)
    lb_all = jnp.concatenate([lp['lin_e_b'] for lp in params['layers']]
                             ).reshape(1, NL * H)

    h = _proj_node(x_p, params['node_w'], params['node_b'])
    ees = _proj_edge(ea_lin, params['edge_w'], params['edge_b'],
                     lw_all, lb_all)

    inv = 1.0 / jnp.sqrt(1.0 + BN_EPS)
    for l, lp in enumerate(params['layers']):
        aggr = _sc_msg(h.reshape(2 * NP, HH), ees[l], src2d, dst2d)
        h = _node_mlp(h, aggr.reshape(2, NP * HH // 128, 128),
                      lp['mlp_w1'], lp['mlp_b1'],
                      lp['mlp_w2'], lp['mlp_b2'],
                      lp['bn_g'] * inv, lp['bn_b'])

    return _pool_head(h, bb, params['head_w1'], params['head_b1'],
                      params['head_w2'], params['head_b2'])


# speedup vs baseline: 1.2426x; 1.0002x over previous
import functools

import jax
import jax.numpy as jnp
from jax import lax
from jax.experimental import pallas as pl
from jax.experimental.pallas import tpu as pltpu, tpu_sc as plsc

N = 50000
E = 800000
G = 64
NODE_IN = 128
EDGE_IN = 16
H = 64
NC_OUT = 5
NL = 3
BN_EPS = 1e-5

NP = 50176
EP = 802816
NBLK = 1024
EBLK = 4096
N_GRID = NP // NBLK
E_GRID = EP // EBLK
PBLK = 7168
P_GRID = NP // PBLK
HH = H // 2

NUM_TILES = 16
CHUNK = 128
CHUNKS_PER_TILE = EP // (NUM_TILES * CHUNK)
STEPS = CHUNKS_PER_TILE // 4
ROWS_PER_TILE = NP // NUM_TILES
EC = CHUNK * HH // 128


def _pack(y):
    ng = y.shape[0] // 1024
    return jnp.concatenate([
        jnp.concatenate([y[t * 1024 + q * 256:t * 1024 + (q + 1) * 256, :]
                         for q in range(4)], axis=1)
        for t in range(ng)], axis=0)


def _unpack(b):
    ng = b.shape[0] // 256
    return jnp.concatenate([
        b[t * 256:(t + 1) * 256, q * 32:(q + 1) * 32]
        for t in range(ng) for q in range(4)], axis=0)


def _proj_node_body(x_ref, w_ref, b_ref, h_ref):
    h = jnp.dot(x_ref[...], w_ref[...], preferred_element_type=jnp.float32)
    h = h + b_ref[...]
    h_ref[0] = _pack(h[:, :HH])
    h_ref[1] = _pack(h[:, HH:])


def _proj_node(x_p, node_w, node_b):
    return pl.pallas_call(
        _proj_node_body,
        grid=(N_GRID,),
        in_specs=[
            pl.BlockSpec((NBLK, NODE_IN), lambda i: (i, 0)),
            pl.BlockSpec((NODE_IN, H), lambda i: (0, 0)),
            pl.BlockSpec((1, H), lambda i: (0, 0)),
        ],
        out_specs=pl.BlockSpec((2, NBLK * HH // 128, 128), lambda i: (0, i, 0)),
        out_shape=jax.ShapeDtypeStruct((2, NP * HH // 128, 128), jnp.float32),
    )(x_p, node_w, node_b.reshape(1, H))


def _proj_edge_body(ea_ref, ew_ref, eb_ref, lw_ref, lb_ref,
                    e0_ref, e1_ref, e2_ref):
    wc = jnp.dot(ew_ref[...], lw_ref[...], preferred_element_type=jnp.float32)
    bc = jnp.dot(eb_ref[...], lw_ref[...],
                 preferred_element_type=jnp.float32) + lb_ref[...]
    ee = jnp.dot(ea_ref[...], wc, preferred_element_type=jnp.float32) + bc
    for l, ref in enumerate((e0_ref, e1_ref, e2_ref)):
        ref[0] = _pack(ee[:, l * H:l * H + HH])
        ref[1] = _pack(ee[:, l * H + HH:(l + 1) * H])


def _proj_edge(ea_lin, edge_w, edge_b, lw_all, lb_all):
    out_sds = jax.ShapeDtypeStruct((2, EP * HH // 128, 128), jnp.float32)
    return pl.pallas_call(
        _proj_edge_body,
        grid=(E_GRID,),
        in_specs=[
            pl.BlockSpec((EBLK, EDGE_IN), lambda i: (i, 0)),
            pl.BlockSpec((EDGE_IN, H), lambda i: (0, 0)),
            pl.BlockSpec((1, H), lambda i: (0, 0)),
            pl.BlockSpec((H, NL * H), lambda i: (0, 0)),
            pl.BlockSpec((1, NL * H), lambda i: (0, 0)),
        ],
        out_specs=[pl.BlockSpec((2, EBLK * HH // 128, 128),
                                lambda i: (0, i, 0))] * NL,
        out_shape=[out_sds] * NL,
    )(ea_lin, edge_w, edge_b.reshape(1, H), lw_all, lb_all)


def _sc_msg_body(h_hbm, ee_hbm, src_hbm, dst_hbm, aggr_hbm,
                 accum, sidx, didx, rows, eebuf, isem, gsem):
    c = lax.axis_index("c")
    s = lax.axis_index("s")
    tile_base = s * ROWS_PER_TILE
    src_off = c * NP

    def zero_row(i, _):
        z = jnp.zeros((16,), jnp.float32)
        rows[0][i, pl.ds(0, 16)] = z
        rows[0][i, pl.ds(16, 16)] = z
        return 0
    lax.fori_loop(0, CHUNK, zero_row, 0)
    zc = 112
    for k in range(ROWS_PER_TILE // zc):
        pltpu.sync_copy(rows[0].at[pl.ds(0, zc)],
                        accum.at[pl.ds(tile_base + k * zc, zc)])
    plsc.subcore_barrier()

    chunk0 = s * CHUNKS_PER_TILE

    def issue_idx(slot, t):
        g = jnp.minimum(chunk0 + t, CHUNKS_PER_TILE * NUM_TILES - 1)
        pltpu.async_copy(src_hbm.at[pl.ds(g, 1)], sidx[slot], isem[slot])
        pltpu.async_copy(dst_hbm.at[pl.ds(g, 1)], didx[slot], isem[slot])

    def wait_idx(slot):
        pltpu.make_async_copy(src_hbm.at[pl.ds(0, 1)], sidx[slot],
                              isem[slot]).wait()
        pltpu.make_async_copy(dst_hbm.at[pl.ds(0, 1)], didx[slot],
                              isem[slot]).wait()

    def issue_gather(slot, islot, t):
        for k in range(8):
            sl = pl.ds(k * 16, 16)
            sidx[islot][0, sl] = sidx[islot][0, sl] + src_off
        pltpu.async_copy(h_hbm.at[sidx[islot].at[0]], rows[slot], gsem[slot])
        te = jnp.minimum(chunk0 + t, CHUNKS_PER_TILE * NUM_TILES - 1)
        pltpu.async_copy(ee_hbm.at[c, pl.ds(te * EC, EC)],
                         eebuf[slot], gsem[slot])

    def wait_gather(slot):
        pltpu.make_async_copy(h_hbm.at[pl.ds(0, CHUNK)], rows[slot],
                              gsem[slot]).wait()
        pltpu.make_async_copy(ee_hbm.at[0, pl.ds(0, EC)], eebuf[slot],
                              gsem[slot]).wait()

    def compute_scatter(slot, islot):
        @plsc.parallel_loop(0, EC, unroll=4)
        def msg_row(r):
            for j in range(8):
                rsl = pl.ds((j % 2) * 16, 16)
                ri = r * 4 + j // 2
                v = eebuf[slot][r, pl.ds(j * 16, 16)] + rows[slot][ri, rsl]
                rows[slot][ri, rsl] = jnp.maximum(v, 0.0)
        pltpu.sync_copy(rows[slot], accum.at[didx[islot].at[0]], add=True)

    for t in range(4):
        issue_idx(t, t)
    wait_idx(0)
    issue_gather(0, 0, 0)

    def step(i, _):
        t0 = 4 * i

        def stage(data_cur, data_nxt, islot_cur, islot_nxt, islot_refill, dt):
            wait_idx(islot_nxt)
            issue_gather(data_nxt, islot_nxt, t0 + dt + 1)
            wait_gather(data_cur)
            compute_scatter(data_cur, islot_cur)
            issue_idx(islot_refill, t0 + dt + 4)

        stage(0, 1, 0, 1, 0, 0)
        stage(1, 0, 1, 2, 1, 1)
        stage(0, 1, 2, 3, 2, 2)
        stage(1, 0, 3, 0, 3, 3)
        return 0

    lax.fori_loop(0, STEPS - 1, step, 0)

    tL = 4 * (STEPS - 1)
    wait_idx(1)
    issue_gather(1, 1, tL + 1)
    wait_gather(0)
    compute_scatter(0, 0)
    wait_idx(2)
    issue_gather(0, 2, tL + 2)
    wait_gather(1)
    compute_scatter(1, 1)
    wait_idx(3)
    issue_gather(1, 3, tL + 3)
    wait_gather(0)
    compute_scatter(0, 2)
    wait_gather(1)
    compute_scatter(1, 3)

    plsc.subcore_barrier()
    pltpu.sync_copy(accum.at[pl.ds(tile_base, ROWS_PER_TILE)],
                    aggr_hbm.at[c, pl.ds(tile_base, ROWS_PER_TILE)])


@jax.jit
def _sc_msg(h_flat, ee, src2d, dst2d):
    mesh = plsc.VectorSubcoreMesh(core_axis_name="c", subcore_axis_name="s",
                                  num_cores=2, num_subcores=NUM_TILES)
    f = functools.partial(
        pl.kernel,
        out_type=jax.ShapeDtypeStruct((2, NP, HH), jnp.float32),
        mesh=mesh,
        scratch_types=[
            pltpu.VMEM_SHARED((NP, HH), jnp.float32),
            [pltpu.VMEM((1, CHUNK), jnp.int32) for _ in range(4)],
            [pltpu.VMEM((1, CHUNK), jnp.int32) for _ in range(4)],
            [pltpu.VMEM((CHUNK, HH), jnp.float32) for _ in range(2)],
            [pltpu.VMEM((EC, 128), jnp.float32) for _ in range(2)],
            [pltpu.SemaphoreType.DMA for _ in range(4)],
            [pltpu.SemaphoreType.DMA for _ in range(2)],
        ],
        compiler_params=pltpu.CompilerParams(use_tc_tiling_on_sc=False),
    )(_sc_msg_body)
    return f(h_flat, ee, src2d, dst2d)


def _mlp_body(h_ref, a_ref, w1_ref, b1_ref, w2_ref, b2_ref, sc_ref, sb_ref,
              out_ref):
    z = jnp.concatenate(
        [_unpack(h_ref[0] + a_ref[0]),
         _unpack(h_ref[1] + a_ref[1])], axis=1)
    t = jnp.maximum(
        jnp.dot(z, w1_ref[...], preferred_element_type=jnp.float32)
        + b1_ref[...], 0.0)
    t = jnp.dot(t, w2_ref[...], preferred_element_type=jnp.float32) + b2_ref[...]
    t = t * sc_ref[...] + sb_ref[...]
    t = jnp.maximum(t, 0.0)
    out_ref[0] = _pack(t[:, :HH])
    out_ref[1] = _pack(t[:, HH:])


def _node_mlp(h, aggr, w1, b1, w2, b2, scale, bias):
    wspec = pl.BlockSpec((H, H), lambda i: (0, 0))
    vspec = pl.BlockSpec((1, H), lambda i: (0, 0))
    lin_spec = pl.BlockSpec((2, NBLK * HH // 128, 128), lambda i: (0, i, 0))
    return pl.pallas_call(
        _mlp_body,
        grid=(N_GRID,),
        in_specs=[lin_spec, lin_spec, wspec, vspec, wspec, vspec, vspec, vspec],
        out_specs=lin_spec,
        out_shape=jax.ShapeDtypeStruct((2, NP * HH // 128, 128), jnp.float32),
    )(h, aggr, w1, b1.reshape(1, H), w2, b2.reshape(1, H),
      scale.reshape(1, H), bias.reshape(1, H))


def _pool_body(h_ref, bb_ref, w1_ref, b1_ref, w2_ref, b2_ref, out_ref,
               sums_ref, maxes_ref, counts_ref):
    i = pl.program_id(0)

    @pl.when(i == 0)
    def _init():
        sums_ref[...] = jnp.zeros_like(sums_ref)
        counts_ref[...] = jnp.zeros_like(counts_ref)
        maxes_ref[...] = jnp.full_like(maxes_ref, -1e30)

    hcat = jnp.concatenate([_unpack(h_ref[0]),
                            _unpack(h_ref[1])], axis=1)
    bb = bb_ref[...][:, :G]
    oh = (bb == lax.broadcasted_iota(jnp.int32, bb.shape, 1)
          .astype(jnp.float32))
    oh = oh.astype(jnp.float32)
    sums_ref[...] += lax.dot_general(
        oh, hcat, (((0,), (0,)), ((), ())),
        preferred_element_type=jnp.float32)
    counts_ref[...] += jnp.sum(oh, axis=0, keepdims=True)
    bmin = jnp.min(bb).astype(jnp.int32)
    bmax = jnp.minimum(jnp.max(bb).astype(jnp.int32), G - 1)

    def seg_max(g, _):
        m = jnp.where(bb[:, :1] == g.astype(jnp.float32), hcat, -1e30)
        mg = jnp.max(m, axis=0, keepdims=True)
        maxes_ref[pl.ds(g, 1), :] = jnp.maximum(maxes_ref[pl.ds(g, 1), :], mg)
        return 0
    lax.fori_loop(bmin, bmax + 1, seg_max, 0)

    @pl.when(i == pl.num_programs(0) - 1)
    def _final():
        counts = counts_ref[...].reshape(G, 1)
        sums = sums_ref[...]
        mean = sums / jnp.maximum(counts, 1.0)
        hmax = jnp.where(counts > 0.0, maxes_ref[...], 0.0)
        feat = jnp.concatenate([mean, hmax, sums], axis=1)
        r = jnp.maximum(
            jnp.dot(feat, w1_ref[...], preferred_element_type=jnp.float32)
            + b1_ref[...], 0.0)
        out_ref[...] = (jnp.dot(r, w2_ref[...],
                                preferred_element_type=jnp.float32)
                        + b2_ref[...])


def _pool_head(h, bb, w1, b1, w2, b2):
    return pl.pallas_call(
        _pool_body,
        grid=(P_GRID,),
        in_specs=[
            pl.BlockSpec((2, PBLK * HH // 128, 128), lambda i: (0, i, 0)),
            pl.BlockSpec((PBLK, 128), lambda i: (i, 0)),
            pl.BlockSpec((3 * H, H), lambda i: (0, 0)),
            pl.BlockSpec((1, H), lambda i: (0, 0)),
            pl.BlockSpec((H, NC_OUT), lambda i: (0, 0)),
            pl.BlockSpec((1, NC_OUT), lambda i: (0, 0)),
        ],
        out_specs=pl.BlockSpec((G, NC_OUT), lambda i: (0, 0)),
        out_shape=jax.ShapeDtypeStruct((G, NC_OUT), jnp.float32),
        scratch_shapes=[
            pltpu.VMEM((G, H), jnp.float32),
            pltpu.VMEM((G, H), jnp.float32),
            pltpu.VMEM((1, G), jnp.float32),
        ],
    )(h, bb, w1, b1.reshape(1, H), w2, b2.reshape(1, NC_OUT))


def _id2flat_vals(v):
    return (v & ~1023) | ((v & 255) << 2) | ((v >> 8) & 3)


def _to_flat_order(a):
    return a.reshape(-1, 4, 256).transpose(0, 2, 1).reshape(a.shape)


def kernel(x, edge_attr, edge_index, batch, params):
    x_p = jnp.pad(x, ((0, NP - N), (0, 0)))
    ea_lin = jnp.pad(edge_attr, ((0, EP - E), (0, 0)))
    src_f = _to_flat_order(_id2flat_vals(jnp.pad(edge_index[0], (0, EP - E))))
    dst_f = _to_flat_order(_id2flat_vals(
        jnp.pad(edge_index[1], (0, EP - E), constant_values=N)))
    src2d = src_f.reshape(EP // CHUNK, CHUNK)
    dst2d = dst_f.reshape(EP // CHUNK, CHUNK)
    bb = jnp.broadcast_to(
        jnp.pad(batch, (0, NP - N), constant_values=G)[:, None].astype(
            jnp.float32), (NP, 128))

    lw_all = jnp.concatenate([lp['lin_e_w'] for lp in params['layers']], axis=1)
    lb_all = jnp.concatenate([lp['lin_e_b'] for lp in params['layers']]
                             ).reshape(1, NL * H)

    h = _proj_node(x_p, params['node_w'], params['node_b'])
    ees = _proj_edge(ea_lin, params['edge_w'], params['edge_b'],
                     lw_all, lb_all)

    inv = 1.0 / jnp.sqrt(1.0 + BN_EPS)
    for l, lp in enumerate(params['layers']):
        aggr = _sc_msg(h.reshape(2 * NP, HH), ees[l], src2d, dst2d)
        h = _node_mlp(h, aggr.reshape(2, NP * HH // 128, 128),
                      lp['mlp_w1'], lp['mlp_b1'],
                      lp['mlp_w2'], lp['mlp_b2'],
                      lp['bn_g'] * inv, lp['bn_b'])

    return _pool_head(h, bb, params['head_w1'], params['head_b1'],
                      params['head_w2'], params['head_b2'])
